# Initial kernel scaffold; baseline (speedup 1.0000x reference)
import jax
import jax.numpy as jnp
from jax.experimental import pallas as pl


def kernel(x, pe):
    raise NotImplementedError("write your pallas kernel here")


# speedup vs baseline: 1.3679x; 1.3679x over previous
import functools

import jax
import jax.numpy as jnp
from jax import lax
from jax.experimental import pallas as pl
from jax.experimental.pallas import tpu as pltpu
from jax.experimental.pallas import tpu_sc as plsc

D_MODEL = 1024
MAX_LEN = 2048

NC = 2
NS = 16
NW = NC * NS

B = 4 * 2048
B_PER_W = B // NW
CHUNK = 64
N_CHUNKS = B_PER_W // CHUNK


def _pe_gather_body(pe_hbm, x_hbm, out_hbm, idx_v, rows_v, sem):
    wid = lax.axis_index("s") * NC + lax.axis_index("c")
    base = wid * B_PER_W
    pltpu.sync_copy(x_hbm.at[pl.ds(base, B_PER_W)], idx_v)
    for c in range(N_CHUNKS):
        pltpu.async_copy(
            pe_hbm.at[idx_v.at[pl.ds(c * CHUNK, CHUNK)]], rows_v, sem
        ).wait()
        pltpu.sync_copy(rows_v, out_hbm.at[pl.ds(base + c * CHUNK, CHUNK)])


@jax.jit
def kernel(x, pe):
    xf = x.reshape(-1).astype(jnp.int32)
    pef = pe.reshape(MAX_LEN, D_MODEL)
    mesh = plsc.VectorSubcoreMesh(core_axis_name="c", subcore_axis_name="s")
    run = pl.kernel(
        _pe_gather_body,
        mesh=mesh,
        out_type=jax.ShapeDtypeStruct((B, D_MODEL), jnp.float32),
        scratch_types=[
            pltpu.VMEM((B_PER_W,), jnp.int32),
            pltpu.VMEM((CHUNK, D_MODEL), jnp.float32),
            pltpu.SemaphoreType.DMA,
        ],
    )
    out = run(pef, xf)
    return out.reshape(x.shape[0], x.shape[1], 1, D_MODEL)


# speedup vs baseline: 1.3809x; 1.0095x over previous
import jax
import jax.numpy as jnp
from jax import lax
from jax.experimental import pallas as pl
from jax.experimental.pallas import tpu as pltpu
from jax.experimental.pallas import tpu_sc as plsc

D_MODEL = 1024
MAX_LEN = 2048

NC = 2
NS = 16
NW = NC * NS

B = 4 * 2048
B_PER_W = B // NW
CHUNK = 32
N_CHUNKS = B_PER_W // CHUNK
NBUF = 3


def _pe_gather_body(pe_hbm, x_hbm, out_hbm, idx_v,
                    buf0, buf1, buf2, sg0, sg1, sg2, ss0, ss1, ss2):
    bufs = (buf0, buf1, buf2)
    sgs = (sg0, sg1, sg2)
    sss = (ss0, ss1, ss2)
    wid = lax.axis_index("s") * NC + lax.axis_index("c")
    base = wid * B_PER_W
    pltpu.sync_copy(x_hbm.at[pl.ds(base, B_PER_W)], idx_v)

    def start_gather(c):
        s = c % NBUF
        return pltpu.async_copy(
            pe_hbm.at[idx_v.at[pl.ds(c * CHUNK, CHUNK)]], bufs[s], sgs[s])

    def start_scatter(c):
        s = c % NBUF
        return pltpu.async_copy(
            bufs[s], out_hbm.at[pl.ds(base + c * CHUNK, CHUNK)], sss[s])

    gathers = [None] * N_CHUNKS
    scats = [None] * N_CHUNKS
    gathers[0] = start_gather(0)
    gathers[1] = start_gather(1)
    for c in range(N_CHUNKS):
        gathers[c].wait()
        scats[c] = start_scatter(c)
        nxt = c + NBUF - 1
        if nxt < N_CHUNKS:
            if c >= 1:
                scats[c - 1].wait()
            gathers[nxt] = start_gather(nxt)
    for c in (N_CHUNKS - 2, N_CHUNKS - 1):
        scats[c].wait()
    scats[N_CHUNKS - 3].wait()


@jax.jit
def kernel(x, pe):
    xf = x.reshape(-1).astype(jnp.int32)
    pef = pe.reshape(MAX_LEN, D_MODEL)
    mesh = plsc.VectorSubcoreMesh(core_axis_name="c", subcore_axis_name="s")
    run = pl.kernel(
        _pe_gather_body,
        mesh=mesh,
        out_type=jax.ShapeDtypeStruct((B, D_MODEL), jnp.float32),
        scratch_types=[
            pltpu.VMEM((B_PER_W,), jnp.int32),
            pltpu.VMEM((CHUNK, D_MODEL), jnp.float32),
            pltpu.VMEM((CHUNK, D_MODEL), jnp.float32),
            pltpu.VMEM((CHUNK, D_MODEL), jnp.float32),
            pltpu.SemaphoreType.DMA,
            pltpu.SemaphoreType.DMA,
            pltpu.SemaphoreType.DMA,
            pltpu.SemaphoreType.DMA,
            pltpu.SemaphoreType.DMA,
            pltpu.SemaphoreType.DMA,
        ],
    )
    out = run(pef, xf)
    return out.reshape(x.shape[0], x.shape[1], 1, D_MODEL)


# speedup vs baseline: 1.3884x; 1.0055x over previous
import jax
import jax.numpy as jnp
from jax import lax
from jax.experimental import pallas as pl
from jax.experimental.pallas import tpu as pltpu
from jax.experimental.pallas import tpu_sc as plsc

D_MODEL = 1024
MAX_LEN = 2048

NC = 2
NS = 16
NW = NC * NS

B = 4 * 2048
B_PER_W = B // NW
CHUNK = 32
N_CHUNKS = B_PER_W // CHUNK
NBUF = 3


def _pe_gather_body(pe_hbm, x_hbm, out_hbm, idx_v,
                    buf0, buf1, buf2, sg0, sg1, sg2, ss0, ss1, ss2):
    bufs = (buf0, buf1, buf2)
    sgs = (sg0, sg1, sg2)
    sss = (ss0, ss1, ss2)
    wid = lax.axis_index("s") * NC + lax.axis_index("c")
    base = wid * B_PER_W
    pltpu.sync_copy(x_hbm.at[pl.ds(base, B_PER_W)], idx_v)

    def start_gather(c):
        s = c % NBUF
        return pltpu.async_copy(
            pe_hbm.at[idx_v.at[pl.ds(c * CHUNK, CHUNK)]], bufs[s], sgs[s])

    def start_scatter(c):
        s = c % NBUF
        return pltpu.async_copy(
            bufs[s], out_hbm.at[pl.ds(base + c * CHUNK, CHUNK)], sss[s])

    gathers = [None] * N_CHUNKS
    scats = [None] * N_CHUNKS
    gathers[0] = start_gather(0)
    gathers[1] = start_gather(1)
    for c in range(N_CHUNKS):
        gathers[c].wait()
        scats[c] = start_scatter(c)
        nxt = c + NBUF - 1
        if nxt < N_CHUNKS:
            if c >= 1:
                scats[c - 1].wait()
            gathers[nxt] = start_gather(nxt)
    for c in (N_CHUNKS - 2, N_CHUNKS - 1):
        scats[c].wait()
    scats[N_CHUNKS - 3].wait()


@jax.jit
def kernel(x, pe):
    xf = x.reshape(-1).astype(jnp.int32)
    pef = pe.reshape(MAX_LEN, D_MODEL)
    mesh = plsc.VectorSubcoreMesh(core_axis_name="c", subcore_axis_name="s")
    run = pl.kernel(
        _pe_gather_body,
        mesh=mesh,
        compiler_params=pltpu.CompilerParams(use_tc_tiling_on_sc=True),
        out_type=jax.ShapeDtypeStruct((B, D_MODEL), jnp.float32),
        scratch_types=[
            pltpu.VMEM((B_PER_W,), jnp.int32),
            pltpu.VMEM((CHUNK, D_MODEL), jnp.float32),
            pltpu.VMEM((CHUNK, D_MODEL), jnp.float32),
            pltpu.VMEM((CHUNK, D_MODEL), jnp.float32),
            pltpu.SemaphoreType.DMA,
            pltpu.SemaphoreType.DMA,
            pltpu.SemaphoreType.DMA,
            pltpu.SemaphoreType.DMA,
            pltpu.SemaphoreType.DMA,
            pltpu.SemaphoreType.DMA,
        ],
    )
    out = run(pef, xf)
    return out.reshape(x.shape[0], x.shape[1], 1, D_MODEL)


# speedup vs baseline: 2.6139x; 1.8826x over previous
import jax
import jax.numpy as jnp
from jax import lax
from jax.experimental import pallas as pl
from jax.experimental.pallas import tpu as pltpu
from jax.experimental.pallas import tpu_sc as plsc

D_MODEL = 1024
MAX_LEN = 2048

NC = 2
NS = 16
NW = NC * NS

BATCH = 4
SEQ = 2048
B = BATCH * SEQ
B_PER_W = B // NW
W_PER_ROW = SEQ // B_PER_W
CHUNK = 32
N_CHUNKS = B_PER_W // CHUNK
NBUF = 3


def _pe_gather_body(pe_hbm, x_hbm, out_hbm, idx_v,
                    buf0, buf1, buf2, sg0, sg1, sg2, ss0, ss1, ss2):
    bufs = (buf0, buf1, buf2)
    sgs = (sg0, sg1, sg2)
    sss = (ss0, ss1, ss2)
    wid = lax.axis_index("s") * NC + lax.axis_index("c")
    b = wid // W_PER_ROW
    off = (wid % W_PER_ROW) * B_PER_W
    pltpu.sync_copy(x_hbm.at[b, pl.ds(off, B_PER_W)], idx_v)

    def start_gather(c):
        s = c % NBUF
        return pltpu.async_copy(
            pe_hbm.at[idx_v.at[pl.ds(c * CHUNK, CHUNK)]], bufs[s], sgs[s])

    def start_scatter(c):
        s = c % NBUF
        return pltpu.async_copy(
            bufs[s], out_hbm.at[b, pl.ds(off + c * CHUNK, CHUNK)], sss[s])

    gathers = [None] * N_CHUNKS
    scats = [None] * N_CHUNKS
    gathers[0] = start_gather(0)
    gathers[1] = start_gather(1)
    for c in range(N_CHUNKS):
        gathers[c].wait()
        scats[c] = start_scatter(c)
        nxt = c + NBUF - 1
        if nxt < N_CHUNKS:
            if c >= 1:
                scats[c - 1].wait()
            gathers[nxt] = start_gather(nxt)
    scats[N_CHUNKS - 3].wait()
    scats[N_CHUNKS - 2].wait()
    scats[N_CHUNKS - 1].wait()


@jax.jit
def kernel(x, pe):
    mesh = plsc.VectorSubcoreMesh(core_axis_name="c", subcore_axis_name="s")
    run = pl.kernel(
        _pe_gather_body,
        mesh=mesh,
        out_type=jax.ShapeDtypeStruct((BATCH, SEQ, 1, D_MODEL), jnp.float32),
        scratch_types=[
            pltpu.VMEM((B_PER_W,), jnp.int32),
            pltpu.VMEM((CHUNK, 1, D_MODEL), jnp.float32),
            pltpu.VMEM((CHUNK, 1, D_MODEL), jnp.float32),
            pltpu.VMEM((CHUNK, 1, D_MODEL), jnp.float32),
            pltpu.SemaphoreType.DMA,
            pltpu.SemaphoreType.DMA,
            pltpu.SemaphoreType.DMA,
            pltpu.SemaphoreType.DMA,
            pltpu.SemaphoreType.DMA,
            pltpu.SemaphoreType.DMA,
        ],
    )
    return run(pe, x.astype(jnp.int32))
